# baseline (speedup 1.0000x reference)
import functools

import jax
import jax.numpy as jnp
from jax import lax
from jax.experimental import pallas as pl
from jax.experimental.pallas import tpu as pltpu
from jax.experimental.pallas import tpu_sc as plsc

N = 10000
D = 128
H = 128
W = 10
E = 320000

_NC = 2
_NS = 16
_LANES = 16
_DH = D // _NC
_EW = E // _NS
_ECHUNK = 400
_ENCHUNK = _EW // _ECHUNK
_NPAD = 10240
_RPT = _NPAD // _NS


def _segsum_body(h_hbm, src_hbm, dst_hbm, zrow_hbm, zdeg_hbm, ones_hbm,
                 agg_hbm, deg_hbm,
                 src_v, dst_v, rows_v, ones_v, acc_sh, hist_sh, sem):
    c = lax.axis_index("c")
    s = lax.axis_index("s")
    pltpu.sync_copy(zrow_hbm, acc_sh.at[pl.ds(s * _RPT, _RPT)])
    pltpu.sync_copy(zdeg_hbm, hist_sh.at[pl.ds(s * _RPT, _RPT)])
    pltpu.sync_copy(ones_hbm, ones_v)
    plsc.subcore_barrier()

    def chunk(i, carry):
        base = s * _EW + i * _ECHUNK
        pltpu.sync_copy(src_hbm.at[pl.ds(base, _ECHUNK)], src_v)
        pltpu.sync_copy(dst_hbm.at[pl.ds(base, _ECHUNK)], dst_v)
        pltpu.async_copy(h_hbm.at[c].at[src_v], rows_v, sem).wait()
        pltpu.sync_copy(rows_v, acc_sh.at[dst_v], add=True)

        @pl.when(c == 0)
        def _():
            pltpu.sync_copy(ones_v, hist_sh.at[dst_v], add=True)

        return carry

    lax.fori_loop(0, _ENCHUNK, chunk, 0)
    plsc.subcore_barrier()
    pltpu.sync_copy(acc_sh.at[pl.ds(s * _RPT, _RPT)],
                    agg_hbm.at[c, pl.ds(s * _RPT, _RPT)])

    @pl.when(c == 0)
    def _():
        pltpu.sync_copy(hist_sh.at[pl.ds(s * _RPT, _RPT)],
                        deg_hbm.at[pl.ds(s * _RPT, _RPT)])


@jax.jit
def _sc_segment_sum(h, src, dst):
    mesh = plsc.VectorSubcoreMesh(core_axis_name="c", subcore_axis_name="s")
    kern = pl.kernel(
        _segsum_body,
        mesh=mesh,
        compiler_params=pltpu.CompilerParams(use_tc_tiling_on_sc=False),
        out_type=(
            jax.ShapeDtypeStruct((_NC, _NPAD, _DH), jnp.float32),
            jax.ShapeDtypeStruct((_NPAD, _LANES), jnp.float32),
        ),
        scratch_types=[
            pltpu.VMEM((_ECHUNK,), jnp.int32),
            pltpu.VMEM((_ECHUNK,), jnp.int32),
            pltpu.VMEM((_ECHUNK, _DH), jnp.float32),
            pltpu.VMEM((_ECHUNK, _LANES), jnp.float32),
            pltpu.VMEM_SHARED((_NPAD, _DH), jnp.float32),
            pltpu.VMEM_SHARED((_NPAD, _LANES), jnp.float32),
            pltpu.SemaphoreType.DMA,
        ],
    )
    hsplit = jnp.stack([h[:, :_DH], h[:, _DH:]])
    zrow = jnp.zeros((_RPT, _DH), jnp.float32)
    zdeg = jnp.zeros((_RPT, _LANES), jnp.float32)
    ones = jnp.ones((_ECHUNK, _LANES), jnp.float32)
    return kern(hsplit, src, dst, zrow, zdeg, ones)


_GCHUNK = 640


def _make_gather_body(nchunk):
    def body(table_hbm, idx_hbm, out_hbm, idx_v, rows_v, sem):
        c = lax.axis_index("c")
        s = lax.axis_index("s")
        w = s * _NC + c
        base = w * (nchunk * _GCHUNK)

        def chunk(i, carry):
            b = base + i * _GCHUNK
            pltpu.sync_copy(idx_hbm.at[pl.ds(b, _GCHUNK)], idx_v)
            pltpu.async_copy(table_hbm.at[idx_v], rows_v, sem).wait()
            pltpu.sync_copy(rows_v, out_hbm.at[pl.ds(b, _GCHUNK)])
            return carry

        lax.fori_loop(0, nchunk, chunk, 0)

    return body


def _sc_gather_rows(table, idx_flat):
    npad = idx_flat.shape[0]
    nchunk = npad // (_NC * _NS * _GCHUNK)
    mesh = plsc.VectorSubcoreMesh(core_axis_name="c", subcore_axis_name="s")
    kern = pl.kernel(
        _make_gather_body(nchunk),
        mesh=mesh,
        out_type=jax.ShapeDtypeStruct((npad, D), jnp.float32),
        scratch_types=[
            pltpu.VMEM((_GCHUNK,), jnp.int32),
            pltpu.VMEM((_GCHUNK, D), jnp.float32),
            pltpu.SemaphoreType.DMA,
        ],
    )
    return kern(table, idx_flat)


def _lstm_steps(x_seq, Wih, Whh, b, keep_seq):
    n = x_seq.shape[0]
    h = jnp.zeros((n, H), jnp.float32)
    c = jnp.zeros((n, H), jnp.float32)
    hs = []
    for t in range(W):
        z = x_seq[:, t, :] @ Wih + h @ Whh + b
        i, f, g, o = jnp.split(z, 4, axis=-1)
        c = jax.nn.sigmoid(f) * c + jax.nn.sigmoid(i) * jnp.tanh(g)
        h = jax.nn.sigmoid(o) * jnp.tanh(c)
        if keep_seq:
            hs.append(h)
    return h, (jnp.stack(hs, axis=1) if keep_seq else None)


def _mlp3(x, W1, b1, W2, b2, W3, b3):
    h = jax.nn.relu(x @ W1 + b1)
    h = jax.nn.relu(h @ W2 + b2)
    return h @ W3 + b3


def _scores_body(hv_ref, bias_ref, out_ref):
    out_ref[...] = hv_ref[...] + bias_ref[...]


def kernel(seq1, neg, tmp, edge_index, msk, samp_bias1, samp_bias2, subgraph, params):
    p = params
    src, dst = edge_index[0], edge_index[1]

    nw = N * W
    blk = _NC * _NS * _GCHUNK
    pad2 = (-2 * nw) % blk
    both_idx = jnp.concatenate(
        [subgraph.reshape(nw), neg.reshape(nw), jnp.zeros((pad2,), jnp.int32)])
    gathered = _sc_gather_rows(seq1, both_idx)
    x_sub = gathered[:nw].reshape(N, W, D)
    x_neg = gathered[nw:2 * nw].reshape(N, W, D)
    h1, hs1 = _lstm_steps(x_sub, p["Wih1"], p["Whh1"], p["b1"], keep_seq=True)
    h2, _ = _lstm_steps(hs1, p["Wih2"], p["Whh2"], p["b2"], keep_seq=False)
    c_out = jax.nn.sigmoid(jnp.mean(h2, axis=0))

    h_neg, _ = _lstm_steps(x_neg, p["Wih1"], p["Whh1"], p["b1"], keep_seq=False)

    t1 = seq1 @ p["Wg1"]
    agg1p, degp = _sc_segment_sum(t1, src, dst)
    degc = jnp.clip(degp[:N, 0], 1.0, None)[:, None]
    f1 = jax.nn.relu(jnp.concatenate([agg1p[0, :N], agg1p[1, :N]], axis=1) / degc)
    t2 = f1 @ p["Wg2"]
    agg2p, _ = _sc_segment_sum(t2, src, dst)
    f2 = jax.nn.relu(jnp.concatenate([agg2p[0, :N], agg2p[1, :N]], axis=1) / degc)

    padg = (-nw) % blk
    tmp_idx = jnp.concatenate([tmp.reshape(nw), jnp.zeros((padg,), jnp.int32)])
    g = _sc_gather_rows(f2, tmp_idx)[:nw].reshape(N, W * D)
    pat = jax.nn.relu(g @ p["Wl1"] + p["bl1"])
    pat = jax.nn.relu(pat @ p["Wl2"] + p["bl2"])

    new_input = jnp.concatenate([h1, seq1, pat], axis=1)
    fea = _mlp3(new_input, p["Wa1"], p["ba1"], p["Wa2"], p["ba2"], p["Wa3"], p["ba3"])

    v = p["Wd"] @ c_out
    hv = jnp.stack([h1 @ v, h_neg @ v], axis=0)
    bias = jnp.stack([samp_bias1, samp_bias2], axis=0)
    scores = pl.pallas_call(
        _scores_body,
        out_shape=jax.ShapeDtypeStruct((2, N), jnp.float32),
    )(hv, bias)
    ret = scores.reshape(2 * N)

    neighbor_sim = jnp.einsum("nd,nwd->nw", seq1, x_sub)
    nb_dec = _mlp3(h1, p["Wls1"], p["bls1"], p["Wls2"], p["bls2"], p["Wls3"], p["bls3"])
    feature_loss3 = jnp.mean((neighbor_sim - nb_dec) ** 2)
    feature_loss = jnp.mean((seq1 - _mlp3(h1, p["Wf1"], p["bf1"], p["Wf2"], p["bf2"], p["Wf3"], p["bf3"])) ** 2)
    feature_loss2 = jnp.mean((seq1 - _mlp3(fea, p["W2f1"], p["b2f1"], p["W2f2"], p["b2f2"], p["W2f3"], p["b2f3"])) ** 2)
    total = feature_loss + feature_loss2 + 1e-07 * feature_loss3
    return ret, total


# speedup vs baseline: 1.0349x; 1.0349x over previous
import functools

import jax
import jax.numpy as jnp
from jax import lax
from jax.experimental import pallas as pl
from jax.experimental.pallas import tpu as pltpu
from jax.experimental.pallas import tpu_sc as plsc

N = 10000
D = 128
H = 128
W = 10
E = 320000

_NC = 2
_NS = 16
_LANES = 16
_DH = D // _NC
_EW = E // _NS
_ECHUNK = 400
_ENCHUNK = _EW // _ECHUNK
_NPAD = 10240
_RPT = _NPAD // _NS


def _segsum_body(h_hbm, src_hbm, dst_hbm, zrow_hbm, zdeg_hbm, ones_hbm,
                 agg_hbm, deg_hbm,
                 src_v0, src_v1, dst_v0, dst_v1, rows_v0, rows_v1, ones_v,
                 acc_sh, hist_sh, sem0, sem1):
    c = lax.axis_index("c")
    s = lax.axis_index("s")
    pltpu.sync_copy(zrow_hbm, acc_sh.at[pl.ds(s * _RPT, _RPT)])
    pltpu.sync_copy(zdeg_hbm, hist_sh.at[pl.ds(s * _RPT, _RPT)])
    pltpu.sync_copy(ones_hbm, ones_v)
    plsc.subcore_barrier()

    def pair(j, carry):
        b0 = s * _EW + (2 * j) * _ECHUNK
        b1 = b0 + _ECHUNK
        pltpu.sync_copy(src_hbm.at[pl.ds(b0, _ECHUNK)], src_v0)
        g0 = pltpu.async_copy(h_hbm.at[c].at[src_v0], rows_v0, sem0)
        pltpu.sync_copy(src_hbm.at[pl.ds(b1, _ECHUNK)], src_v1)
        g1 = pltpu.async_copy(h_hbm.at[c].at[src_v1], rows_v1, sem1)
        pltpu.sync_copy(dst_hbm.at[pl.ds(b0, _ECHUNK)], dst_v0)
        g0.wait()
        pltpu.sync_copy(rows_v0, acc_sh.at[dst_v0], add=True)

        @pl.when(c == 0)
        def _():
            pltpu.sync_copy(ones_v, hist_sh.at[dst_v0], add=True)

        pltpu.sync_copy(dst_hbm.at[pl.ds(b1, _ECHUNK)], dst_v1)
        g1.wait()
        pltpu.sync_copy(rows_v1, acc_sh.at[dst_v1], add=True)

        @pl.when(c == 0)
        def _():
            pltpu.sync_copy(ones_v, hist_sh.at[dst_v1], add=True)

        return carry

    lax.fori_loop(0, _ENCHUNK // 2, pair, 0)
    plsc.subcore_barrier()
    pltpu.sync_copy(acc_sh.at[pl.ds(s * _RPT, _RPT)],
                    agg_hbm.at[c, pl.ds(s * _RPT, _RPT)])

    @pl.when(c == 0)
    def _():
        pltpu.sync_copy(hist_sh.at[pl.ds(s * _RPT, _RPT)],
                        deg_hbm.at[pl.ds(s * _RPT, _RPT)])


@jax.jit
def _sc_segment_sum(h, src, dst):
    mesh = plsc.VectorSubcoreMesh(core_axis_name="c", subcore_axis_name="s")
    kern = pl.kernel(
        _segsum_body,
        mesh=mesh,
        compiler_params=pltpu.CompilerParams(use_tc_tiling_on_sc=False),
        out_type=(
            jax.ShapeDtypeStruct((_NC, _NPAD, _DH), jnp.float32),
            jax.ShapeDtypeStruct((_NPAD, _LANES), jnp.float32),
        ),
        scratch_types=[
            pltpu.VMEM((_ECHUNK,), jnp.int32),
            pltpu.VMEM((_ECHUNK,), jnp.int32),
            pltpu.VMEM((_ECHUNK,), jnp.int32),
            pltpu.VMEM((_ECHUNK,), jnp.int32),
            pltpu.VMEM((_ECHUNK, _DH), jnp.float32),
            pltpu.VMEM((_ECHUNK, _DH), jnp.float32),
            pltpu.VMEM((_ECHUNK, _LANES), jnp.float32),
            pltpu.VMEM_SHARED((_NPAD, _DH), jnp.float32),
            pltpu.VMEM_SHARED((_NPAD, _LANES), jnp.float32),
            pltpu.SemaphoreType.DMA,
            pltpu.SemaphoreType.DMA,
        ],
    )
    hsplit = jnp.stack([h[:, :_DH], h[:, _DH:]])
    zrow = jnp.zeros((_RPT, _DH), jnp.float32)
    zdeg = jnp.zeros((_RPT, _LANES), jnp.float32)
    ones = jnp.ones((_ECHUNK, _LANES), jnp.float32)
    return kern(hsplit, src, dst, zrow, zdeg, ones)


_GCHUNK = 400


def _make_gather_body(nchunk):
    def body(table_hbm, idx_hbm, out_hbm,
             idx_v0, idx_v1, rows_v0, rows_v1, sem0, sem1):
        c = lax.axis_index("c")
        s = lax.axis_index("s")
        w = s * _NC + c
        base = w * (nchunk * _GCHUNK)
        idx_b = (idx_v0, idx_v1)
        row_b = (rows_v0, rows_v1)
        sems = (sem0, sem1)
        handles = [None, None]
        pltpu.sync_copy(idx_hbm.at[pl.ds(base, _GCHUNK)], idx_b[0])
        handles[0] = pltpu.async_copy(table_hbm.at[idx_b[0]], row_b[0], sems[0])
        for i in range(nchunk):
            cur = i % 2
            nxt = (i + 1) % 2
            if i + 1 < nchunk:
                b = base + (i + 1) * _GCHUNK
                pltpu.sync_copy(idx_hbm.at[pl.ds(b, _GCHUNK)], idx_b[nxt])
                handles[nxt] = pltpu.async_copy(
                    table_hbm.at[idx_b[nxt]], row_b[nxt], sems[nxt])
            handles[cur].wait()
            pltpu.sync_copy(row_b[cur],
                            out_hbm.at[pl.ds(base + i * _GCHUNK, _GCHUNK)])

    return body


def _sc_gather_rows(table, idx_flat):
    npad = idx_flat.shape[0]
    nchunk = npad // (_NC * _NS * _GCHUNK)
    mesh = plsc.VectorSubcoreMesh(core_axis_name="c", subcore_axis_name="s")
    kern = pl.kernel(
        _make_gather_body(nchunk),
        mesh=mesh,
        out_type=jax.ShapeDtypeStruct((npad, D), jnp.float32),
        scratch_types=[
            pltpu.VMEM((_GCHUNK,), jnp.int32),
            pltpu.VMEM((_GCHUNK,), jnp.int32),
            pltpu.VMEM((_GCHUNK, D), jnp.float32),
            pltpu.VMEM((_GCHUNK, D), jnp.float32),
            pltpu.SemaphoreType.DMA,
            pltpu.SemaphoreType.DMA,
        ],
    )
    return kern(table, idx_flat)


def _lstm_steps(x_seq, Wih, Whh, b, keep_seq):
    n = x_seq.shape[0]
    h = jnp.zeros((n, H), jnp.float32)
    c = jnp.zeros((n, H), jnp.float32)
    hs = []
    for t in range(W):
        z = x_seq[:, t, :] @ Wih + h @ Whh + b
        i, f, g, o = jnp.split(z, 4, axis=-1)
        c = jax.nn.sigmoid(f) * c + jax.nn.sigmoid(i) * jnp.tanh(g)
        h = jax.nn.sigmoid(o) * jnp.tanh(c)
        if keep_seq:
            hs.append(h)
    return h, (jnp.stack(hs, axis=1) if keep_seq else None)


def _mlp3(x, W1, b1, W2, b2, W3, b3):
    h = jax.nn.relu(x @ W1 + b1)
    h = jax.nn.relu(h @ W2 + b2)
    return h @ W3 + b3


def _scores_body(hv_ref, bias_ref, out_ref):
    out_ref[...] = hv_ref[...] + bias_ref[...]


def kernel(seq1, neg, tmp, edge_index, msk, samp_bias1, samp_bias2, subgraph, params):
    p = params
    src, dst = edge_index[0], edge_index[1]

    nw = N * W
    blk = _NC * _NS * _GCHUNK
    pad2 = (-2 * nw) % blk
    both_idx = jnp.concatenate(
        [subgraph.reshape(nw), neg.reshape(nw), jnp.zeros((pad2,), jnp.int32)])
    gathered = _sc_gather_rows(seq1, both_idx)
    x_sub = gathered[:nw].reshape(N, W, D)
    x_neg = gathered[nw:2 * nw].reshape(N, W, D)
    h1, hs1 = _lstm_steps(x_sub, p["Wih1"], p["Whh1"], p["b1"], keep_seq=True)
    h2, _ = _lstm_steps(hs1, p["Wih2"], p["Whh2"], p["b2"], keep_seq=False)
    c_out = jax.nn.sigmoid(jnp.mean(h2, axis=0))

    h_neg, _ = _lstm_steps(x_neg, p["Wih1"], p["Whh1"], p["b1"], keep_seq=False)

    t1 = seq1 @ p["Wg1"]
    agg1p, degp = _sc_segment_sum(t1, src, dst)
    degc = jnp.clip(degp[:N, 0], 1.0, None)[:, None]
    f1 = jax.nn.relu(jnp.concatenate([agg1p[0, :N], agg1p[1, :N]], axis=1) / degc)
    t2 = f1 @ p["Wg2"]
    agg2p, _ = _sc_segment_sum(t2, src, dst)
    f2 = jax.nn.relu(jnp.concatenate([agg2p[0, :N], agg2p[1, :N]], axis=1) / degc)

    padg = (-nw) % blk
    tmp_idx = jnp.concatenate([tmp.reshape(nw), jnp.zeros((padg,), jnp.int32)])
    g = _sc_gather_rows(f2, tmp_idx)[:nw].reshape(N, W * D)
    pat = jax.nn.relu(g @ p["Wl1"] + p["bl1"])
    pat = jax.nn.relu(pat @ p["Wl2"] + p["bl2"])

    new_input = jnp.concatenate([h1, seq1, pat], axis=1)
    fea = _mlp3(new_input, p["Wa1"], p["ba1"], p["Wa2"], p["ba2"], p["Wa3"], p["ba3"])

    v = p["Wd"] @ c_out
    hv = jnp.stack([h1 @ v, h_neg @ v], axis=0)
    bias = jnp.stack([samp_bias1, samp_bias2], axis=0)
    scores = pl.pallas_call(
        _scores_body,
        out_shape=jax.ShapeDtypeStruct((2, N), jnp.float32),
    )(hv, bias)
    ret = scores.reshape(2 * N)

    neighbor_sim = jnp.einsum("nd,nwd->nw", seq1, x_sub)
    nb_dec = _mlp3(h1, p["Wls1"], p["bls1"], p["Wls2"], p["bls2"], p["Wls3"], p["bls3"])
    feature_loss3 = jnp.mean((neighbor_sim - nb_dec) ** 2)
    feature_loss = jnp.mean((seq1 - _mlp3(h1, p["Wf1"], p["bf1"], p["Wf2"], p["bf2"], p["Wf3"], p["bf3"])) ** 2)
    feature_loss2 = jnp.mean((seq1 - _mlp3(fea, p["W2f1"], p["b2f1"], p["W2f2"], p["b2f2"], p["W2f3"], p["b2f3"])) ** 2)
    total = feature_loss + feature_loss2 + 1e-07 * feature_loss3
    return ret, total


# speedup vs baseline: 1.8369x; 1.7751x over previous
import functools

import jax
import jax.numpy as jnp
from jax import lax
from jax.experimental import pallas as pl
from jax.experimental.pallas import tpu as pltpu
from jax.experimental.pallas import tpu_sc as plsc

N = 10000
D = 128
H = 128
W = 10
E = 320000

_NC = 2
_NS = 16
_LANES = 16
_DH = D // _NC
_EW = E // _NS
_ECHUNK = 400
_ENCHUNK = _EW // _ECHUNK
_NPAD = 10240
_RPT = _NPAD // _NS


def _segsum_body(h_hbm, src_hbm, dst_hbm, zrow_hbm, zdeg_hbm, ones_hbm,
                 agg_hbm, deg_hbm,
                 src_v0, src_v1, dst_v0, dst_v1, rows_v0, rows_v1, ones_v,
                 acc_sh, hist_sh, sem0, sem1):
    c = lax.axis_index("c")
    s = lax.axis_index("s")
    pltpu.sync_copy(zrow_hbm, acc_sh.at[pl.ds(s * _RPT, _RPT)])
    pltpu.sync_copy(zdeg_hbm, hist_sh.at[pl.ds(s * _RPT, _RPT)])
    pltpu.sync_copy(ones_hbm, ones_v)
    plsc.subcore_barrier()

    def pair(j, carry):
        b0 = s * _EW + (2 * j) * _ECHUNK
        b1 = b0 + _ECHUNK
        pltpu.sync_copy(src_hbm.at[pl.ds(b0, _ECHUNK)], src_v0)
        g0 = pltpu.async_copy(h_hbm.at[c].at[src_v0], rows_v0, sem0)
        pltpu.sync_copy(src_hbm.at[pl.ds(b1, _ECHUNK)], src_v1)
        g1 = pltpu.async_copy(h_hbm.at[c].at[src_v1], rows_v1, sem1)
        pltpu.sync_copy(dst_hbm.at[pl.ds(b0, _ECHUNK)], dst_v0)
        g0.wait()
        pltpu.sync_copy(rows_v0, acc_sh.at[dst_v0], add=True)

        @pl.when(c == 0)
        def _():
            pltpu.sync_copy(ones_v, hist_sh.at[dst_v0], add=True)

        pltpu.sync_copy(dst_hbm.at[pl.ds(b1, _ECHUNK)], dst_v1)
        g1.wait()
        pltpu.sync_copy(rows_v1, acc_sh.at[dst_v1], add=True)

        @pl.when(c == 0)
        def _():
            pltpu.sync_copy(ones_v, hist_sh.at[dst_v1], add=True)

        return carry

    lax.fori_loop(0, _ENCHUNK // 2, pair, 0)
    plsc.subcore_barrier()
    pltpu.sync_copy(acc_sh.at[pl.ds(s * _RPT, _RPT)],
                    agg_hbm.at[c, pl.ds(s * _RPT, _RPT)])

    @pl.when(c == 0)
    def _():
        pltpu.sync_copy(hist_sh.at[pl.ds(s * _RPT, _RPT)],
                        deg_hbm.at[pl.ds(s * _RPT, _RPT)])


@jax.jit
def _sc_segment_sum(h, src, dst):
    mesh = plsc.VectorSubcoreMesh(core_axis_name="c", subcore_axis_name="s")
    kern = pl.kernel(
        _segsum_body,
        mesh=mesh,
        compiler_params=pltpu.CompilerParams(use_tc_tiling_on_sc=False),
        out_type=(
            jax.ShapeDtypeStruct((_NC, _NPAD, _DH), jnp.float32),
            jax.ShapeDtypeStruct((_NPAD, _LANES), jnp.float32),
        ),
        scratch_types=[
            pltpu.VMEM((_ECHUNK,), jnp.int32),
            pltpu.VMEM((_ECHUNK,), jnp.int32),
            pltpu.VMEM((_ECHUNK,), jnp.int32),
            pltpu.VMEM((_ECHUNK,), jnp.int32),
            pltpu.VMEM((_ECHUNK, _DH), jnp.float32),
            pltpu.VMEM((_ECHUNK, _DH), jnp.float32),
            pltpu.VMEM((_ECHUNK, _LANES), jnp.float32),
            pltpu.VMEM_SHARED((_NPAD, _DH), jnp.float32),
            pltpu.VMEM_SHARED((_NPAD, _LANES), jnp.float32),
            pltpu.SemaphoreType.DMA,
            pltpu.SemaphoreType.DMA,
        ],
    )
    hsplit = jnp.stack([h[:, :_DH], h[:, _DH:]])
    zrow = jnp.zeros((_RPT, _DH), jnp.float32)
    zdeg = jnp.zeros((_RPT, _LANES), jnp.float32)
    ones = jnp.ones((_ECHUNK, _LANES), jnp.float32)
    return kern(hsplit, src, dst, zrow, zdeg, ones)


_GCHUNK = 400


def _make_gather_body(nchunk):
    def body(table_hbm, idx_hbm, out_hbm,
             idx_v0, idx_v1, rows_v0, rows_v1, sem0, sem1):
        c = lax.axis_index("c")
        s = lax.axis_index("s")
        w = s * _NC + c
        base = w * (nchunk * _GCHUNK)
        idx_b = (idx_v0, idx_v1)
        row_b = (rows_v0, rows_v1)
        sems = (sem0, sem1)
        handles = [None, None]
        pltpu.sync_copy(idx_hbm.at[pl.ds(base, _GCHUNK)], idx_b[0])
        handles[0] = pltpu.async_copy(table_hbm.at[idx_b[0]], row_b[0], sems[0])
        for i in range(nchunk):
            cur = i % 2
            nxt = (i + 1) % 2
            if i + 1 < nchunk:
                b = base + (i + 1) * _GCHUNK
                pltpu.sync_copy(idx_hbm.at[pl.ds(b, _GCHUNK)], idx_b[nxt])
                handles[nxt] = pltpu.async_copy(
                    table_hbm.at[idx_b[nxt]], row_b[nxt], sems[nxt])
            handles[cur].wait()
            pltpu.sync_copy(row_b[cur],
                            out_hbm.at[pl.ds(base + i * _GCHUNK, _GCHUNK)])

    return body


def _sc_gather_rows(table, idx_flat):
    npad = idx_flat.shape[0]
    nchunk = npad // (_NC * _NS * _GCHUNK)
    mesh = plsc.VectorSubcoreMesh(core_axis_name="c", subcore_axis_name="s")
    kern = pl.kernel(
        _make_gather_body(nchunk),
        mesh=mesh,
        out_type=jax.ShapeDtypeStruct((npad, D), jnp.float32),
        scratch_types=[
            pltpu.VMEM((_GCHUNK,), jnp.int32),
            pltpu.VMEM((_GCHUNK,), jnp.int32),
            pltpu.VMEM((_GCHUNK, D), jnp.float32),
            pltpu.VMEM((_GCHUNK, D), jnp.float32),
            pltpu.SemaphoreType.DMA,
            pltpu.SemaphoreType.DMA,
        ],
    )
    return kern(table, idx_flat)


_LB = 400
_LGRID = N // _LB


def _make_lstm_body(lb):
  def _lstm_tc_body(xs_ref, xn_ref, s1_ref, wih1, whh1, wih2, whh2, b1r, b2r,
                  wg1, h1_o, hn_o, h2s_o, ns_o, t1_o):
    i = pl.program_id(0)
    s1 = s1_ref[...]
    zero = jnp.zeros((lb, H), jnp.float32)
    h1 = c1 = h2 = c2 = hn = cn = zero
    ns_cols = []
    for t in range(W):
        xt = xs_ref[:, t, :]
        z = xt @ wih1[...] + h1 @ whh1[...] + b1r[...]
        ig = jax.nn.sigmoid(z[:, 0:H])
        fg = jax.nn.sigmoid(z[:, H:2 * H])
        gg = jnp.tanh(z[:, 2 * H:3 * H])
        og = jax.nn.sigmoid(z[:, 3 * H:4 * H])
        c1 = fg * c1 + ig * gg
        h1 = og * jnp.tanh(c1)

        z2 = h1 @ wih2[...] + h2 @ whh2[...] + b2r[...]
        ig2 = jax.nn.sigmoid(z2[:, 0:H])
        fg2 = jax.nn.sigmoid(z2[:, H:2 * H])
        gg2 = jnp.tanh(z2[:, 2 * H:3 * H])
        og2 = jax.nn.sigmoid(z2[:, 3 * H:4 * H])
        c2 = fg2 * c2 + ig2 * gg2
        h2 = og2 * jnp.tanh(c2)

        xnt = xn_ref[:, t, :]
        zn = xnt @ wih1[...] + hn @ whh1[...] + b1r[...]
        ign = jax.nn.sigmoid(zn[:, 0:H])
        fgn = jax.nn.sigmoid(zn[:, H:2 * H])
        ggn = jnp.tanh(zn[:, 2 * H:3 * H])
        ogn = jax.nn.sigmoid(zn[:, 3 * H:4 * H])
        cn = fgn * cn + ign * ggn
        hn = ogn * jnp.tanh(cn)

        ns_cols.append(jnp.sum(s1 * xt, axis=1, keepdims=True))

    h1_o[...] = h1
    hn_o[...] = hn
    ns_o[...] = jnp.concatenate(
        ns_cols + [jnp.zeros((lb, D - W), jnp.float32)], axis=1)
    t1_o[...] = s1 @ wg1[...]

    @pl.when(i == 0)
    def _():
        h2s_o[...] = jnp.zeros((1, H), jnp.float32)

    h2s_o[...] += jnp.sum(h2, axis=0, keepdims=True)

  return _lstm_tc_body


def _lstm_tc(x_sub, x_neg, seq1, p, interpret=False):
    n = x_sub.shape[0]
    lb = _LB if n % _LB == 0 else n
    grid = n // lb
    out_shape = (
        jax.ShapeDtypeStruct((n, H), jnp.float32),
        jax.ShapeDtypeStruct((n, H), jnp.float32),
        jax.ShapeDtypeStruct((1, H), jnp.float32),
        jax.ShapeDtypeStruct((n, D), jnp.float32),
        jax.ShapeDtypeStruct((n, D), jnp.float32),
    )
    full = lambda shp: pl.BlockSpec(shp, lambda i: (0,) * len(shp))
    row_blk = pl.BlockSpec((lb, H), lambda i: (i, 0))
    return pl.pallas_call(
        _make_lstm_body(lb),
        grid=(grid,),
        in_specs=[
            pl.BlockSpec((lb, W, D), lambda i: (i, 0, 0)),
            pl.BlockSpec((lb, W, D), lambda i: (i, 0, 0)),
            row_blk,
            full((D, 4 * H)), full((H, 4 * H)), full((H, 4 * H)),
            full((H, 4 * H)), full((1, 4 * H)), full((1, 4 * H)),
            full((D, D)),
        ],
        out_specs=(
            row_blk, row_blk, pl.BlockSpec((1, H), lambda i: (0, 0)),
            row_blk, row_blk,
        ),
        out_shape=out_shape,
        interpret=interpret,
    )(x_sub, x_neg, seq1,
      p["Wih1"], p["Whh1"], p["Wih2"], p["Whh2"],
      p["b1"].reshape(1, 4 * H), p["b2"].reshape(1, 4 * H), p["Wg1"])


def _lstm_steps(x_seq, Wih, Whh, b, keep_seq):
    n = x_seq.shape[0]
    h = jnp.zeros((n, H), jnp.float32)
    c = jnp.zeros((n, H), jnp.float32)
    hs = []
    for t in range(W):
        z = x_seq[:, t, :] @ Wih + h @ Whh + b
        i, f, g, o = jnp.split(z, 4, axis=-1)
        c = jax.nn.sigmoid(f) * c + jax.nn.sigmoid(i) * jnp.tanh(g)
        h = jax.nn.sigmoid(o) * jnp.tanh(c)
        if keep_seq:
            hs.append(h)
    return h, (jnp.stack(hs, axis=1) if keep_seq else None)


def _mlp3(x, W1, b1, W2, b2, W3, b3):
    h = jax.nn.relu(x @ W1 + b1)
    h = jax.nn.relu(h @ W2 + b2)
    return h @ W3 + b3


def _scores_body(hv_ref, bias_ref, out_ref):
    out_ref[...] = hv_ref[...] + bias_ref[...]


def kernel(seq1, neg, tmp, edge_index, msk, samp_bias1, samp_bias2, subgraph, params):
    p = params
    src, dst = edge_index[0], edge_index[1]

    nw = N * W
    blk = _NC * _NS * _GCHUNK
    pad2 = (-2 * nw) % blk
    both_idx = jnp.concatenate(
        [subgraph.reshape(nw), neg.reshape(nw), jnp.zeros((pad2,), jnp.int32)])
    gathered = _sc_gather_rows(seq1, both_idx)
    x_sub = gathered[:nw].reshape(N, W, D)
    x_neg = gathered[nw:2 * nw].reshape(N, W, D)
    h1, h_neg, h2sum, ns_pad, t1 = _lstm_tc(x_sub, x_neg, seq1, p)
    c_out = jax.nn.sigmoid(h2sum[0] / N)
    agg1p, degp = _sc_segment_sum(t1, src, dst)
    degc = jnp.clip(degp[:N, 0], 1.0, None)[:, None]
    f1 = jax.nn.relu(jnp.concatenate([agg1p[0, :N], agg1p[1, :N]], axis=1) / degc)
    t2 = f1 @ p["Wg2"]
    agg2p, _ = _sc_segment_sum(t2, src, dst)
    f2 = jax.nn.relu(jnp.concatenate([agg2p[0, :N], agg2p[1, :N]], axis=1) / degc)

    padg = (-nw) % blk
    tmp_idx = jnp.concatenate([tmp.reshape(nw), jnp.zeros((padg,), jnp.int32)])
    g = _sc_gather_rows(f2, tmp_idx)[:nw].reshape(N, W * D)
    pat = jax.nn.relu(g @ p["Wl1"] + p["bl1"])
    pat = jax.nn.relu(pat @ p["Wl2"] + p["bl2"])

    new_input = jnp.concatenate([h1, seq1, pat], axis=1)
    fea = _mlp3(new_input, p["Wa1"], p["ba1"], p["Wa2"], p["ba2"], p["Wa3"], p["ba3"])

    v = p["Wd"] @ c_out
    hv = jnp.stack([h1 @ v, h_neg @ v], axis=0)
    bias = jnp.stack([samp_bias1, samp_bias2], axis=0)
    scores = pl.pallas_call(
        _scores_body,
        out_shape=jax.ShapeDtypeStruct((2, N), jnp.float32),
    )(hv, bias)
    ret = scores.reshape(2 * N)

    neighbor_sim = ns_pad[:, :W]
    nb_dec = _mlp3(h1, p["Wls1"], p["bls1"], p["Wls2"], p["bls2"], p["Wls3"], p["bls3"])
    feature_loss3 = jnp.mean((neighbor_sim - nb_dec) ** 2)
    feature_loss = jnp.mean((seq1 - _mlp3(h1, p["Wf1"], p["bf1"], p["Wf2"], p["bf2"], p["Wf3"], p["bf3"])) ** 2)
    feature_loss2 = jnp.mean((seq1 - _mlp3(fea, p["W2f1"], p["b2f1"], p["W2f2"], p["b2f2"], p["W2f3"], p["b2f3"])) ** 2)
    total = feature_loss + feature_loss2 + 1e-07 * feature_loss3
    return ret, total
